# Initial kernel scaffold; baseline (speedup 1.0000x reference)
import jax
import jax.numpy as jnp
from jax.experimental import pallas as pl


def kernel(collab_bev_data_list, bandwidth_budget, utility_map_list):
    raise NotImplementedError("write your pallas kernel here")


# speedup vs baseline: 33.9246x; 33.9246x over previous
import jax
import jax.numpy as jnp
from jax import lax
from jax.experimental import pallas as pl
from jax.experimental.pallas import tpu as pltpu

_CV, _CF, _CD = 64, 128, 32
_NCH = _CV + _CF + _CD
_B = 4
_HW = 64 * 64
_CBLK = 32
_NBLK = _NCH // _CBLK
_BND0 = _CV // _CBLK
_BND1 = (_CV + _CF) // _CBLK
_COSTS = (4.0, 2.0, 1.0)


def _select_and_mask_body(budget_ref, u0_ref, u1_ref, u2_ref, col_ref,
                          out_ref, sel_ref):
    j = pl.program_id(0)

    @pl.when(j == 0)
    def _compute_selection():
        u0 = u0_ref[...]
        u1 = u1_ref[...]
        u2 = u2_ref[...]
        best = jnp.maximum(u0, jnp.maximum(u1, u2))
        g = jnp.where(u0 == best, 0, jnp.where(u1 == best, 1, 2)).astype(jnp.int32)
        cost = jnp.where(g == 0, _COSTS[0],
                         jnp.where(g == 1, _COSTS[1], _COSTS[2])).astype(jnp.float32)
        valid = best > 0.0
        vc = jnp.where(valid, cost, 0.0)
        h = lax.bitcast_convert_type(jnp.maximum(best, 0.0), jnp.int32)
        lidx = (_HW - 1) - lax.broadcasted_iota(jnp.int32, (_B, _HW), 1)
        T = budget_ref[0, 0]
        tgt = [T - _COSTS[0], T - _COSTS[1], T - _COSTS[2]]

        def asum(mid):
            return jnp.sum(jnp.where(h > mid, vc, 0.0), axis=1, keepdims=True)

        lo0 = jnp.full((_B, 1), -1, jnp.int32)
        hi0 = jnp.full((_B, 1), 1 << 30, jnp.int32)

        def h_body(_, carry):
            los, his = carry
            nlo, nhi = [], []
            for c in range(3):
                mid = los[c] + (his[c] - los[c]) // 2
                ok = asum(mid) <= tgt[c]
                nhi.append(jnp.where(ok, mid, his[c]))
                nlo.append(jnp.where(ok, los[c], mid))
            return tuple(nlo), tuple(nhi)

        _, hstar = lax.fori_loop(0, 31, h_body, ((lo0,) * 3, (hi0,) * 3))
        aat = [asum(hstar[c]) for c in range(3)]

        def bsum(c, mid):
            return jnp.sum(jnp.where((h == hstar[c]) & (lidx > mid), vc, 0.0),
                           axis=1, keepdims=True)

        llo0 = jnp.full((_B, 1), -1, jnp.int32)
        lhi0 = jnp.full((_B, 1), _HW - 1, jnp.int32)

        def l_body(_, carry):
            los_, his_ = carry
            nlo, nhi = [], []
            for c in range(3):
                mid = los_[c] + (his_[c] - los_[c]) // 2
                ok = aat[c] + bsum(c, mid) <= tgt[c]
                nhi.append(jnp.where(ok, mid, his_[c]))
                nlo.append(jnp.where(ok, los_[c], mid))
            return tuple(nlo), tuple(nhi)

        _, lstar = lax.fori_loop(0, 13, l_body, ((llo0,) * 3, (lhi0,) * 3))

        hs = jnp.where(g == 0, hstar[0], jnp.where(g == 1, hstar[1], hstar[2]))
        ls = jnp.where(g == 0, lstar[0], jnp.where(g == 1, lstar[1], lstar[2]))
        acc1 = valid & ((h > hs) | ((h == hs) & (lidx >= ls)))

        used = jnp.sum(jnp.where(acc1, cost, 0.0), axis=1, keepdims=True)
        r = T - used
        m = valid & jnp.logical_not(acc1)
        ph = jnp.max(jnp.where(m, h, -1), axis=1, keepdims=True)
        plv = jnp.max(jnp.where(m & (h == ph), lidx, -1), axis=1, keepdims=True)

        ext = jnp.zeros((_B, _HW), jnp.bool_)
        for _t in range(3):
            cand = (m & jnp.logical_not(ext) & (cost <= r)
                    & ((h < ph) | ((h == ph) & (lidx < plv))))
            ch = jnp.max(jnp.where(cand, h, -1), axis=1, keepdims=True)
            cl = jnp.max(jnp.where(cand & (h == ch), lidx, -1),
                         axis=1, keepdims=True)
            newext = cand & (h == ch) & (lidx == cl)
            ext = ext | newext
            r = r - jnp.sum(jnp.where(newext, cost, 0.0), axis=1, keepdims=True)
            found = ch >= 0
            ph = jnp.where(found, ch, ph)
            plv = jnp.where(found, cl, plv)

        acc = acc1 | ext
        sel_ref[...] = jnp.where(acc, g.astype(jnp.float32), -1.0)

    s = sel_ref[...]
    t = jnp.where(j < _BND0, 0.0, jnp.where(j < _BND1, 1.0, 2.0))
    mask = jnp.where(s == t, 1.0, 0.0)
    out_ref[...] = col_ref[...] * mask[:, None, :]


def _specs():
    in_specs = [
        pl.BlockSpec(memory_space=pltpu.SMEM),
        pl.BlockSpec((_B, _HW), lambda j: (0, 0)),
        pl.BlockSpec((_B, _HW), lambda j: (0, 0)),
        pl.BlockSpec((_B, _HW), lambda j: (0, 0)),
        pl.BlockSpec((_B, _CBLK, _HW), lambda j: (0, j, 0)),
    ]
    out_specs = [
        pl.BlockSpec((_B, _CBLK, _HW), lambda j: (0, j, 0)),
        pl.BlockSpec((_B, _HW), lambda j: (0, 0)),
    ]
    out_shape = [
        jax.ShapeDtypeStruct((_B, _NCH, _HW), jnp.float32),
        jax.ShapeDtypeStruct((_B, _HW), jnp.float32),
    ]
    return in_specs, out_specs, out_shape


def kernel(collab_bev_data_list, bandwidth_budget, utility_map_list):
    col3 = collab_bev_data_list.reshape(_B, _NCH, _HW)
    u = utility_map_list.reshape(_B, _HW, 3)
    budget = (jnp.asarray(bandwidth_budget, jnp.float32) / _B).reshape(1, 1)
    in_specs, out_specs, out_shape = _specs()
    out3, sel = pl.pallas_call(
        _select_and_mask_body,
        grid=(_NBLK,),
        in_specs=in_specs,
        out_specs=out_specs,
        out_shape=out_shape,
    )(budget, u[:, :, 0], u[:, :, 1], u[:, :, 2], col3)
    return out3.reshape(_B, _NCH, 64, 64), sel.reshape(_B, 64, 64)
